# baseline (speedup 1.0000x reference)
import functools

import jax
import jax.numpy as jnp
from jax import lax
from jax.experimental import pallas as pl
from jax.experimental.pallas import tpu as pltpu
from jax.experimental.pallas import tpu_sc as plsc

_CHUNK = 8


def _sc_kernel(x_hbm, t_hbm, o_hbm, xv, tv, nw):
    info = plsc.get_sparse_core_info()
    nc = info.num_cores
    seq_len, batch, d = x_hbm.shape
    rows_per_w = seq_len // nw
    nchunks = rows_per_w // _CHUNK
    lanegroups = d // 16

    wid = lax.axis_index("s") * nc + lax.axis_index("c")
    w_base = wid * rows_per_w

    def chunk_body(c, carry):
        base = w_base + c * _CHUNK
        pltpu.sync_copy(x_hbm.at[pl.ds(base, _CHUNK)], xv)
        pltpu.sync_copy(t_hbm.at[pl.ds(base, _CHUNK)], tv)

        def add_body(j, carry2):
            r = j >> 6
            k = (j & (lanegroups - 1)) * 16
            tvec = tv[r, pl.ds(k, 16)]
            for b in range(batch):
                xv[r, b, pl.ds(k, 16)] = xv[r, b, pl.ds(k, 16)] + tvec
            return carry2

        lax.fori_loop(0, _CHUNK * lanegroups, add_body, 0)
        pltpu.sync_copy(xv, o_hbm.at[pl.ds(base, _CHUNK)])
        return carry

    lax.fori_loop(0, nchunks, chunk_body, 0)


def kernel(x, table):
    seq_len, batch, d = x.shape
    info = plsc.get_sparse_core_info()
    nw = info.num_cores * info.num_subcores
    mesh = plsc.VectorSubcoreMesh(core_axis_name="c", subcore_axis_name="s")
    k = functools.partial(
        pl.kernel,
        mesh=mesh,
        out_type=jax.ShapeDtypeStruct((seq_len, batch, d), x.dtype),
        scratch_types=[
            pltpu.VMEM((_CHUNK, batch, d), x.dtype),
            pltpu.VMEM((_CHUNK, d), table.dtype),
        ],
    )(functools.partial(_sc_kernel, nw=nw))
    return k(x, table)


# speedup vs baseline: 3.0546x; 3.0546x over previous
import jax
import jax.numpy as jnp
from jax.experimental import pallas as pl
from jax.experimental.pallas import tpu as pltpu

_BS = 64
_NBUF = 16


def _pipelined_kernel(x_hbm, t_hbm, o_hbm, xb, tb, ob, sx, st, so):
    seq_len, batch, _ = x_hbm.shape
    nsteps = seq_len // _BS

    def in_copies(i):
        slot = i % _NBUF
        return (
            pltpu.make_async_copy(
                x_hbm.at[pl.ds(i * _BS, _BS)], xb.at[slot], sx.at[slot]),
            pltpu.make_async_copy(
                t_hbm.at[pl.ds(i * _BS, _BS)], tb.at[slot], st.at[slot]),
        )

    def out_copy(i):
        slot = i % _NBUF
        return pltpu.make_async_copy(
            ob.at[slot], o_hbm.at[pl.ds(i * _BS, _BS)], so.at[slot])

    for i in range(min(_NBUF, nsteps)):
        for c in in_copies(i):
            c.start()

    for i in range(nsteps):
        slot = i % _NBUF
        for c in in_copies(i):
            c.wait()
        if i >= _NBUF:
            out_copy(i - _NBUF).wait()
        t = tb[slot]
        for b in range(batch):
            ob[slot, :, b, :] = xb[slot, :, b, :] + t
        out_copy(i).start()
        if i + _NBUF < nsteps:
            for c in in_copies(i + _NBUF):
                c.start()

    for i in range(max(0, nsteps - _NBUF), nsteps):
        out_copy(i).wait()


def kernel(x, table):
    seq_len, batch, d = x.shape
    return pl.pallas_call(
        _pipelined_kernel,
        in_specs=[
            pl.BlockSpec(memory_space=pl.ANY),
            pl.BlockSpec(memory_space=pl.ANY),
        ],
        out_specs=pl.BlockSpec(memory_space=pl.ANY),
        out_shape=jax.ShapeDtypeStruct((seq_len, batch, d), x.dtype),
        scratch_shapes=[
            pltpu.VMEM((_NBUF, _BS, batch, d), x.dtype),
            pltpu.VMEM((_NBUF, _BS, d), table.dtype),
            pltpu.VMEM((_NBUF, _BS, batch, d), x.dtype),
            pltpu.SemaphoreType.DMA((_NBUF,)),
            pltpu.SemaphoreType.DMA((_NBUF,)),
            pltpu.SemaphoreType.DMA((_NBUF,)),
        ],
    )(x, table)
